# baseline (speedup 1.0000x reference)
import functools

import jax
import jax.numpy as jnp
from jax import lax
from jax.experimental import pallas as pl
from jax.experimental.pallas import tpu as pltpu
from jax.experimental.pallas import tpu_sc as plsc


def _make_gather(V, D, N):
    info = plsc.get_sparse_core_info()
    NC, NS = info.num_cores, info.num_subcores
    NW = NC * NS
    assert N % NW == 0
    n_per_w = N // NW
    C = 2560
    assert n_per_w % C == 0
    n_chunks = n_per_w // C

    mesh = plsc.VectorSubcoreMesh(core_axis_name="c", subcore_axis_name="s")

    @functools.partial(
        pl.kernel,
        mesh=mesh,
        compiler_params=pltpu.CompilerParams(use_tc_tiling_on_sc=False),
        out_type=jax.ShapeDtypeStruct((N, D), jnp.float32),
        scratch_types=[
            pltpu.VMEM((C,), jnp.int32),
            pltpu.VMEM((C, D), jnp.float32),
            pltpu.SemaphoreType.DMA,
        ],
    )
    def gather_kernel(table_hbm, idx_hbm, out_hbm, idx_v, rows_v, sem):
        wid = lax.axis_index("s") * NC + lax.axis_index("c")
        base = wid * n_per_w

        def body(i, carry):
            off = base + i * C
            pltpu.sync_copy(idx_hbm.at[pl.ds(off, C)], idx_v)
            pltpu.async_copy(table_hbm.at[idx_v], rows_v, sem).wait()
            pltpu.sync_copy(rows_v, out_hbm.at[pl.ds(off, C)])
            return carry

        lax.fori_loop(0, n_chunks, body, 0)

    return gather_kernel


def kernel(table, x):
    V, D = table.shape
    B, H = x.shape
    N = B * H
    idx = x.reshape(N)
    out = _make_gather(V, D, N)(table, idx)
    return out.reshape(B, H, D)


# speedup vs baseline: 1.0115x; 1.0115x over previous
import functools

import jax
import jax.numpy as jnp
from jax import lax
from jax.experimental import pallas as pl
from jax.experimental.pallas import tpu as pltpu
from jax.experimental.pallas import tpu_sc as plsc


def _make_gather(V, D, N):
    info = plsc.get_sparse_core_info()
    NC, NS = info.num_cores, info.num_subcores
    NW = NC * NS
    assert N % NW == 0
    n_per_w = N // NW
    C = 2560
    NBUF = 2
    assert n_per_w % (C * NBUF) == 0
    n_chunks = n_per_w // C
    n_outer = n_chunks // NBUF

    mesh = plsc.VectorSubcoreMesh(core_axis_name="c", subcore_axis_name="s")

    @functools.partial(
        pl.kernel,
        mesh=mesh,
        compiler_params=pltpu.CompilerParams(use_tc_tiling_on_sc=False),
        out_type=jax.ShapeDtypeStruct((N, D), jnp.float32),
        scratch_types=[
            pltpu.VMEM((n_per_w,), jnp.int32),
            *[pltpu.VMEM((C, D), jnp.float32) for _ in range(NBUF)],
            *[pltpu.SemaphoreType.DMA for _ in range(2 * NBUF)],
        ],
    )
    def gather_kernel(table_hbm, idx_hbm, out_hbm, idx_full, *rest):
        rows = rest[:NBUF]
        gsem = rest[NBUF : 2 * NBUF]
        wsem = rest[2 * NBUF :]
        wid = lax.axis_index("s") * NC + lax.axis_index("c")
        base = wid * n_per_w

        pltpu.sync_copy(idx_hbm.at[pl.ds(base, n_per_w)], idx_full)

        for b in range(NBUF):
            pltpu.async_copy(
                table_hbm.at[idx_full.at[pl.ds(b * C, C)]], rows[b], gsem[b]
            )

        def body(g, carry):
            for b in range(NBUF):
                i = g * NBUF + b
                pltpu.make_async_copy(
                    table_hbm.at[pl.ds(0, C)], rows[b], gsem[b]
                ).wait()
                pltpu.async_copy(
                    rows[b], out_hbm.at[pl.ds(base + i * C, C)], wsem[b]
                )

                @pl.when(g < n_outer - 1)
                def _():
                    pltpu.make_async_copy(
                        rows[b], out_hbm.at[pl.ds(base, C)], wsem[b]
                    ).wait()
                    pltpu.async_copy(
                        table_hbm.at[
                            idx_full.at[pl.ds(((g + 1) * NBUF + b) * C, C)]
                        ],
                        rows[b],
                        gsem[b],
                    )

            return carry

        lax.fori_loop(0, n_outer, body, 0)

        for b in range(NBUF):
            pltpu.make_async_copy(
                rows[b], out_hbm.at[pl.ds(base, C)], wsem[b]
            ).wait()

    return gather_kernel


def kernel(table, x):
    V, D = table.shape
    B, H = x.shape
    N = B * H
    idx = x.reshape(N)
    out = _make_gather(V, D, N)(table, idx)
    return out.reshape(B, H, D)


# speedup vs baseline: 1.2846x; 1.2700x over previous
import functools

import jax
import jax.numpy as jnp
from jax import lax
from jax.experimental import pallas as pl
from jax.experimental.pallas import tpu as pltpu
from jax.experimental.pallas import tpu_sc as plsc


def _make_gather(V, D, B, H):
    info = plsc.get_sparse_core_info()
    NC, NS = info.num_cores, info.num_subcores
    NW = NC * NS
    assert B % NW == 0
    rows_per_w = B // NW
    RC = 32
    NBUF = 2
    assert rows_per_w % (RC * NBUF) == 0
    n_chunks = rows_per_w // RC
    n_outer = n_chunks // NBUF

    mesh = plsc.VectorSubcoreMesh(core_axis_name="c", subcore_axis_name="s")

    @functools.partial(
        pl.kernel,
        mesh=mesh,
        compiler_params=pltpu.CompilerParams(use_tc_tiling_on_sc=False),
        out_type=jax.ShapeDtypeStruct((B, H, D), jnp.float32),
        scratch_types=[
            pltpu.VMEM((rows_per_w, H), jnp.int32),
            *[pltpu.VMEM((RC, H, D), jnp.float32) for _ in range(NBUF)],
            *[pltpu.SemaphoreType.DMA for _ in range(2 * NBUF)],
        ],
    )
    def gather_kernel(table_hbm, x_hbm, out_hbm, idx_full, *rest):
        rows = rest[:NBUF]
        gsem = rest[NBUF : 2 * NBUF]
        wsem = rest[2 * NBUF :]
        wid = lax.axis_index("s") * NC + lax.axis_index("c")
        base_row = wid * rows_per_w

        pltpu.sync_copy(x_hbm.at[pl.ds(base_row, rows_per_w)], idx_full)

        def start_gather(chunk, b):
            for r in range(RC):
                pltpu.async_copy(
                    table_hbm.at[idx_full.at[chunk * RC + r]],
                    rows[b].at[r],
                    gsem[b],
                )

        def wait_gather(b):
            pltpu.make_async_copy(
                out_hbm.at[pl.ds(0, RC)], rows[b], gsem[b]
            ).wait()

        def start_write(chunk, b):
            pltpu.async_copy(
                rows[b], out_hbm.at[pl.ds(base_row + chunk * RC, RC)], wsem[b]
            )

        def wait_write(b):
            pltpu.make_async_copy(
                rows[b], out_hbm.at[pl.ds(base_row, RC)], wsem[b]
            ).wait()

        for b in range(NBUF):
            start_gather(b, b)

        def body(g, carry):
            for b in range(NBUF):
                i = g * NBUF + b
                wait_gather(b)
                start_write(i, b)

                @pl.when(g < n_outer - 1)
                def _():
                    wait_write(b)
                    start_gather((g + 1) * NBUF + b, b)

            return carry

        lax.fori_loop(0, n_outer, body, 0)

        for b in range(NBUF):
            wait_write(b)

    return gather_kernel


def kernel(table, x):
    V, D = table.shape
    B, H = x.shape
    return _make_gather(V, D, B, H)(table, x)


# speedup vs baseline: 1.5624x; 1.2163x over previous
import functools

import jax
import jax.numpy as jnp
from jax import lax
from jax.experimental import pallas as pl
from jax.experimental.pallas import tpu as pltpu
from jax.experimental.pallas import tpu_sc as plsc


def _make_gather(V, D, B, H):
    info = plsc.get_sparse_core_info()
    NC, NS, L = info.num_cores, info.num_subcores, info.num_lanes
    NW = NC * NS
    assert B % (NW * 128) == 0
    rows_per_w = B // NW
    RC = 32
    NBUF = 2
    assert rows_per_w % (RC * NBUF) == 0
    n_chunks = rows_per_w // RC
    n_outer = n_chunks // NBUF
    DT = D // 8
    BT = B // 128
    bt_per_chunk_div = 128 // RC

    mesh = plsc.VectorSubcoreMesh(core_axis_name="c", subcore_axis_name="s")

    @functools.partial(
        pl.kernel,
        mesh=mesh,
        compiler_params=pltpu.CompilerParams(
            use_tc_tiling_on_sc=False, needs_layout_passes=False
        ),
        out_type=jax.ShapeDtypeStruct((H, DT, BT, 8, 128), jnp.float32),
        scratch_types=[
            pltpu.VMEM((rows_per_w, H), jnp.int32),
            *[pltpu.VMEM((RC, H, D), jnp.float32) for _ in range(NBUF)],
            *[pltpu.VMEM((H, DT, 8, RC), jnp.float32) for _ in range(NBUF)],
            *[pltpu.SemaphoreType.DMA for _ in range(2 * NBUF)],
        ],
    )
    def gather_kernel(table_hbm, x_hbm, out_hbm, idx_full, *rest):
        rows = rest[:NBUF]
        tbuf = rest[NBUF : 2 * NBUF]
        gsem = rest[2 * NBUF : 3 * NBUF]
        wsem = rest[3 * NBUF :]
        wid = lax.axis_index("s") * NC + lax.axis_index("c")
        base_row = wid * rows_per_w
        base_bt = wid * (rows_per_w // 128)

        pltpu.sync_copy(x_hbm.at[pl.ds(base_row, rows_per_w)], idx_full)

        def start_gather(chunk, b):
            for r in range(RC):
                pltpu.async_copy(
                    table_hbm.at[idx_full.at[chunk * RC + r]],
                    rows[b].at[r],
                    gsem[b],
                )

        def wait_gather(b):
            for r in range(RC):
                pltpu.make_async_copy(
                    table_hbm.at[pl.ds(0, H)], rows[b].at[r], gsem[b]
                ).wait()

        def transpose_chunk(b):
            bvecs = [
                lax.iota(jnp.int32, L) + k * L for k in range(RC // L)
            ]

            def h_body(h, carry):
                hvec = jnp.full((L,), h, jnp.int32)
                for k in range(RC // L):
                    for d in range(D):
                        v = plsc.load_gather(
                            rows[b],
                            [bvecs[k], hvec, jnp.full((L,), d, jnp.int32)],
                        )
                        tbuf[b][h, d // 8, d % 8, pl.ds(k * L, L)] = v
                return carry

            lax.fori_loop(0, H, h_body, 0)

        def write_slices(chunk, b):
            bti = base_bt + chunk // bt_per_chunk_div
            col0 = (chunk % bt_per_chunk_div) * RC
            return out_hbm.at[:, :, bti, :, pl.ds(col0, RC)]

        def start_write(chunk, b):
            pltpu.async_copy(tbuf[b], write_slices(chunk, b), wsem[b])

        def wait_write(b):
            pltpu.make_async_copy(
                tbuf[b], out_hbm.at[:, :, 0, :, pl.ds(0, RC)], wsem[b]
            ).wait()

        for b in range(NBUF):
            start_gather(b, b)

        def body(g, carry):
            for b in range(NBUF):
                i = g * NBUF + b
                wait_gather(b)

                @pl.when(g > 0)
                def _():
                    wait_write(b)

                transpose_chunk(b)
                start_write(i, b)

                @pl.when(g < n_outer - 1)
                def _():
                    start_gather((g + 1) * NBUF + b, b)

            return carry

        lax.fori_loop(0, n_outer, body, 0)

        for b in range(NBUF):
            wait_write(b)

    return gather_kernel


def kernel(table, x):
    V, D = table.shape
    B, H = x.shape
    out5 = _make_gather(V, D, B, H)(table, x)
    return out5.transpose(2, 4, 0, 1, 3).reshape(B, H, D)


# speedup vs baseline: 1.7055x; 1.0916x over previous
import functools

import jax
import jax.numpy as jnp
from jax import lax
from jax.experimental import pallas as pl
from jax.experimental.pallas import tpu as pltpu
from jax.experimental.pallas import tpu_sc as plsc


def _make_gather(V, D, B, H):
    info = plsc.get_sparse_core_info()
    NC, NS, L = info.num_cores, info.num_subcores, info.num_lanes
    NW = NC * NS
    assert B % (NW * 128) == 0
    rows_per_w = B // NW
    DT = D // 8
    BT = B // 128
    bt_per_w = rows_per_w // 128
    NBUF = 2
    assert H % NBUF == 0

    mesh = plsc.VectorSubcoreMesh(core_axis_name="c", subcore_axis_name="s")

    @functools.partial(
        pl.kernel,
        mesh=mesh,
        compiler_params=pltpu.CompilerParams(
            use_tc_tiling_on_sc=False, needs_layout_passes=False
        ),
        out_type=jax.ShapeDtypeStruct((H, DT, BT, 8, 128), jnp.float32),
        scratch_types=[
            pltpu.VMEM((rows_per_w, H), jnp.int32),
            pltpu.VMEM((H, rows_per_w), jnp.int32),
            *[pltpu.VMEM((rows_per_w, D), jnp.float32) for _ in range(NBUF)],
            *[pltpu.VMEM((DT, bt_per_w, 8, 128), jnp.float32) for _ in range(NBUF)],
            *[pltpu.SemaphoreType.DMA for _ in range(2 * NBUF)],
        ],
    )
    def gather_kernel(table_hbm, x_hbm, out_hbm, idx_full, idx_t, *rest):
        rows = rest[:NBUF]
        tbuf = rest[NBUF : 2 * NBUF]
        gsem = rest[2 * NBUF : 3 * NBUF]
        wsem = rest[3 * NBUF :]
        wid = lax.axis_index("s") * NC + lax.axis_index("c")
        base_row = wid * rows_per_w
        bt0 = wid * bt_per_w

        pltpu.sync_copy(x_hbm.at[pl.ds(base_row, rows_per_w)], idx_full)

        def idx_t_body(h, carry):
            hvec = jnp.full((L,), h, jnp.int32)
            for m in range(rows_per_w // L):
                bvec = lax.iota(jnp.int32, L) + m * L
                v = plsc.load_gather(idx_full, [bvec, hvec])
                idx_t[h, pl.ds(m * L, L)] = v
            return carry

        lax.fori_loop(0, H, idx_t_body, 0)

        def start_gather(h, b):
            pltpu.async_copy(table_hbm.at[idx_t.at[h]], rows[b], gsem[b])

        def wait_gather(b):
            pltpu.make_async_copy(
                table_hbm.at[pl.ds(0, rows_per_w)], rows[b], gsem[b]
            ).wait()

        def transpose_slab(b):
            for bt in range(bt_per_w):
                for jg in range(128 // L):
                    bvec = lax.iota(jnp.int32, L) + (bt * 128 + jg * L)
                    for d in range(D):
                        v = plsc.load_gather(
                            rows[b], [bvec, jnp.full((L,), d, jnp.int32)]
                        )
                        tbuf[b][d // 8, bt, d % 8, pl.ds(jg * L, L)] = v

        def start_write(h, b):
            pltpu.async_copy(
                tbuf[b], out_hbm.at[h, :, pl.ds(bt0, bt_per_w)], wsem[b]
            )

        def wait_write(b):
            pltpu.make_async_copy(
                tbuf[b], out_hbm.at[0, :, pl.ds(0, bt_per_w)], wsem[b]
            ).wait()

        for b in range(NBUF):
            start_gather(b, b)

        def body(g, carry):
            for b in range(NBUF):
                h = g * NBUF + b
                wait_gather(b)

                @pl.when(g > 0)
                def _():
                    wait_write(b)

                transpose_slab(b)
                start_write(h, b)

                @pl.when(g < H // NBUF - 1)
                def _():
                    start_gather(h + NBUF, b)

            return carry

        lax.fori_loop(0, H // NBUF, body, 0)

        for b in range(NBUF):
            wait_write(b)

    return gather_kernel


def kernel(table, x):
    V, D = table.shape
    B, H = x.shape
    out5 = _make_gather(V, D, B, H)(table, x)
    return out5.transpose(2, 4, 0, 1, 3).reshape(B, H, D)


# speedup vs baseline: 2.1767x; 1.2763x over previous
import functools

import jax
import jax.numpy as jnp
from jax import lax
from jax.experimental import pallas as pl
from jax.experimental.pallas import tpu as pltpu
from jax.experimental.pallas import tpu_sc as plsc


def _make_gather(V, D, B, H):
    info = plsc.get_sparse_core_info()
    NC, NS, L = info.num_cores, info.num_subcores, info.num_lanes
    NW = NC * NS
    assert B % (NW * 128) == 0
    rows_per_w = B // NW
    DT = D // 8
    BT = B // 128
    bt_per_w = rows_per_w // 128
    NBUF = 2
    assert H % NBUF == 0

    mesh = plsc.VectorSubcoreMesh(core_axis_name="c", subcore_axis_name="s")

    @functools.partial(
        pl.kernel,
        mesh=mesh,
        compiler_params=pltpu.CompilerParams(
            use_tc_tiling_on_sc=False, needs_layout_passes=False
        ),
        out_type=jax.ShapeDtypeStruct((H, DT, BT, 8, 128), jnp.float32),
        scratch_types=[
            pltpu.VMEM((rows_per_w, H), jnp.int32),
            pltpu.VMEM((H, rows_per_w), jnp.int32),
            *[pltpu.VMEM((rows_per_w, D), jnp.float32) for _ in range(NBUF)],
            *[pltpu.VMEM((DT, bt_per_w, 8, 128), jnp.float32) for _ in range(NBUF)],
            *[pltpu.SemaphoreType.DMA for _ in range(2 * NBUF)],
        ],
    )
    def gather_kernel(table_hbm, x_hbm, out_hbm, idx_full, idx_t, *rest):
        rows = rest[:NBUF]
        tbuf = rest[NBUF : 2 * NBUF]
        gsem = rest[2 * NBUF : 3 * NBUF]
        wsem = rest[3 * NBUF :]
        wid = lax.axis_index("s") * NC + lax.axis_index("c")
        base_row = wid * rows_per_w
        bt0 = wid * bt_per_w

        pltpu.sync_copy(x_hbm.at[pl.ds(base_row, rows_per_w)], idx_full)

        def idx_t_body(h, carry):
            hvec = jnp.full((L,), h, jnp.int32)
            vs = [
                plsc.load_gather(
                    idx_full, [lax.iota(jnp.int32, L) + m * L, hvec]
                )
                for m in range(rows_per_w // L)
            ]
            for m in range(rows_per_w // L):
                idx_t[h, pl.ds(m * L, L)] = vs[m]
            return carry

        lax.fori_loop(0, H, idx_t_body, 0)

        def start_gather(h, b):
            pltpu.async_copy(table_hbm.at[idx_t.at[h]], rows[b], gsem[b])

        def wait_gather(b):
            pltpu.make_async_copy(
                table_hbm.at[pl.ds(0, rows_per_w)], rows[b], gsem[b]
            ).wait()

        def transpose_slab(b):
            for bt in range(bt_per_w):
                for jg in range(128 // L):
                    bvec = lax.iota(jnp.int32, L) + (bt * 128 + jg * L)
                    vs = [
                        plsc.load_gather(
                            rows[b], [bvec, jnp.full((L,), d, jnp.int32)]
                        )
                        for d in range(D)
                    ]
                    for d in range(D):
                        tbuf[b][d // 8, bt, d % 8, pl.ds(jg * L, L)] = vs[d]

        def start_write(h, b):
            pltpu.async_copy(
                tbuf[b], out_hbm.at[h, :, pl.ds(bt0, bt_per_w)], wsem[b]
            )

        def wait_write(b):
            pltpu.make_async_copy(
                tbuf[b], out_hbm.at[0, :, pl.ds(0, bt_per_w)], wsem[b]
            ).wait()

        for b in range(NBUF):
            start_gather(b, b)

        def body(g, carry):
            for b in range(NBUF):
                h = g * NBUF + b
                wait_gather(b)

                @pl.when(g > 0)
                def _():
                    wait_write(b)

                transpose_slab(b)
                start_write(h, b)

                @pl.when(g < H // NBUF - 1)
                def _():
                    start_gather(h + NBUF, b)

            return carry

        lax.fori_loop(0, H // NBUF, body, 0)

        for b in range(NBUF):
            wait_write(b)

    return gather_kernel


def kernel(table, x):
    V, D = table.shape
    B, H = x.shape
    out5 = _make_gather(V, D, B, H)(table, x)
    return out5.transpose(2, 4, 0, 1, 3).reshape(B, H, D)


# speedup vs baseline: 2.1955x; 1.0086x over previous
import functools

import jax
import jax.numpy as jnp
from jax import lax
from jax.experimental import pallas as pl
from jax.experimental.pallas import tpu as pltpu
from jax.experimental.pallas import tpu_sc as plsc


def _make_gather(V, D, B, H):
    info = plsc.get_sparse_core_info()
    NC, NS, L = info.num_cores, info.num_subcores, info.num_lanes
    NW = NC * NS
    assert B % (NW * 128) == 0
    rows_per_w = B // NW
    DT = D // 8
    BT = B // 128
    bt_per_w = rows_per_w // 128
    NBUF = 2
    assert H % NBUF == 0

    mesh = plsc.VectorSubcoreMesh(core_axis_name="c", subcore_axis_name="s")

    @functools.partial(
        pl.kernel,
        mesh=mesh,
        compiler_params=pltpu.CompilerParams(
            use_tc_tiling_on_sc=False, needs_layout_passes=False
        ),
        out_type=jax.ShapeDtypeStruct((H, DT, BT, 8, 128), jnp.float32),
        scratch_types=[
            pltpu.VMEM((H, rows_per_w), jnp.int32),
            *[pltpu.VMEM((rows_per_w, D), jnp.float32) for _ in range(NBUF)],
            *[pltpu.VMEM((DT, bt_per_w, 8, 128), jnp.float32) for _ in range(NBUF)],
            *[pltpu.SemaphoreType.DMA for _ in range(2 * NBUF)],
        ],
    )
    def gather_kernel(table_hbm, xt_hbm, out_hbm, idx_t, *rest):
        rows = rest[:NBUF]
        tbuf = rest[NBUF : 2 * NBUF]
        gsem = rest[2 * NBUF : 3 * NBUF]
        wsem = rest[3 * NBUF :]
        wid = lax.axis_index("s") * NC + lax.axis_index("c")
        base_row = wid * rows_per_w
        bt0 = wid * bt_per_w

        pltpu.sync_copy(xt_hbm.at[:, pl.ds(base_row, rows_per_w)], idx_t)

        def start_gather(h, b):
            pltpu.async_copy(table_hbm.at[idx_t.at[h]], rows[b], gsem[b])

        def wait_gather(b):
            pltpu.make_async_copy(
                table_hbm.at[pl.ds(0, rows_per_w)], rows[b], gsem[b]
            ).wait()

        def transpose_slab(b):
            for bt in range(bt_per_w):
                for jg in range(128 // L):
                    bvec = lax.iota(jnp.int32, L) + (bt * 128 + jg * L)
                    vs = [
                        plsc.load_gather(
                            rows[b], [bvec, jnp.full((L,), d, jnp.int32)]
                        )
                        for d in range(D)
                    ]
                    for d in range(D):
                        tbuf[b][d // 8, bt, d % 8, pl.ds(jg * L, L)] = vs[d]

        def start_write(h, b):
            pltpu.async_copy(
                tbuf[b], out_hbm.at[h, :, pl.ds(bt0, bt_per_w)], wsem[b]
            )

        def wait_write(b):
            pltpu.make_async_copy(
                tbuf[b], out_hbm.at[0, :, pl.ds(0, bt_per_w)], wsem[b]
            ).wait()

        for b in range(NBUF):
            start_gather(b, b)

        def body(g, carry):
            for b in range(NBUF):
                h = g * NBUF + b
                wait_gather(b)

                @pl.when(g > 0)
                def _():
                    wait_write(b)

                transpose_slab(b)
                start_write(h, b)

                @pl.when(g < H // NBUF - 1)
                def _():
                    start_gather(h + NBUF, b)

            return carry

        lax.fori_loop(0, H // NBUF, body, 0)

        for b in range(NBUF):
            wait_write(b)

    return gather_kernel


def kernel(table, x):
    V, D = table.shape
    B, H = x.shape
    out5 = _make_gather(V, D, B, H)(table, x.T)
    return out5.transpose(2, 4, 0, 1, 3).reshape(B, H, D)
